# Initial kernel scaffold; baseline (speedup 1.0000x reference)
import jax
import jax.numpy as jnp
from jax.experimental import pallas as pl


def kernel(x, edge_index, batch, W1, b1, W2, b2):
    raise NotImplementedError("write your pallas kernel here")


# speedup vs baseline: 14.2184x; 14.2184x over previous
import functools

import jax
import jax.numpy as jnp
from jax import lax
from jax.experimental import pallas as pl
from jax.experimental.pallas import tpu as pltpu
from jax.experimental.pallas import tpu_sc as plsc

_NCORES = 2
_NSUB = 16
_NTILES = _NCORES * _NSUB
_CHUNK = 128
_BR = 512
_NG = 128


def _sc_mesh():
    return plsc.VectorSubcoreMesh(
        core_axis_name="c", subcore_axis_name="s",
        num_cores=_NCORES, num_subcores=_NSUB)


def _sc_degree(dst_t, ones_w, zeros_f, rpad, width):
    nch = dst_t.shape[1]
    rpt = rpad // _NSUB

    @functools.partial(
        pl.kernel,
        out_type=jax.ShapeDtypeStruct((_NCORES, rpad, width), jnp.float32),
        mesh=_sc_mesh(),
        scratch_types=[
            pltpu.VMEM((1, _CHUNK), jnp.int32),
            pltpu.VMEM((_CHUNK, width), jnp.float32),
            pltpu.VMEM_SHARED((rpad, width), jnp.float32),
        ],
    )
    def k(dst_hbm, ones_hbm, z_hbm, out_hbm, idx_v, ones_v, deg_sh):
        core = lax.axis_index("c")
        sid = lax.axis_index("s")
        gid = core * _NSUB + sid
        r0 = sid * rpt
        pltpu.sync_copy(ones_hbm, ones_v)
        pltpu.sync_copy(z_hbm.at[pl.ds(r0, rpt)], deg_sh.at[pl.ds(r0, rpt)])
        plsc.subcore_barrier()

        @pl.loop(0, nch)
        def _(c):
            pltpu.sync_copy(dst_hbm.at[gid, c], idx_v)
            pltpu.sync_copy(ones_v, deg_sh.at[idx_v.at[0]], add=True)

        plsc.subcore_barrier()
        pltpu.sync_copy(deg_sh.at[pl.ds(r0, rpt)],
                        out_hbm.at[core, pl.ds(r0, rpt)])

    return k(dst_t, ones_w, zeros_f)


def _sc_edge_pass(hs, src_t, dst_t, zeros_f, rpad):
    nch = src_t.shape[1]
    rpt = rpad // _NSUB
    feat = hs.shape[1]

    @functools.partial(
        pl.kernel,
        out_type=jax.ShapeDtypeStruct((_NCORES, rpad, feat), jnp.float32),
        mesh=_sc_mesh(),
        scratch_types=[
            pltpu.VMEM((1, _CHUNK), jnp.int32),
            pltpu.VMEM((1, _CHUNK), jnp.int32),
            pltpu.VMEM((_CHUNK, feat), jnp.float32),
            pltpu.VMEM_SHARED((rpad, feat), jnp.float32),
            pltpu.SemaphoreType.DMA,
        ],
    )
    def k(hs_hbm, src_hbm, dst_hbm, z_hbm, out_hbm,
          sidx, didx, rows, acc_sh, sem):
        core = lax.axis_index("c")
        sid = lax.axis_index("s")
        gid = core * _NSUB + sid
        r0 = sid * rpt
        pltpu.sync_copy(z_hbm.at[pl.ds(r0, rpt)], acc_sh.at[pl.ds(r0, rpt)])
        plsc.subcore_barrier()

        @pl.loop(0, nch)
        def _(c):
            pltpu.sync_copy(src_hbm.at[gid, c], sidx)
            pltpu.sync_copy(dst_hbm.at[gid, c], didx)
            pltpu.async_copy(hs_hbm.at[sidx.at[0]], rows, sem).wait()
            pltpu.sync_copy(rows, acc_sh.at[didx.at[0]], add=True)

        plsc.subcore_barrier()
        pltpu.sync_copy(acc_sh.at[pl.ds(r0, rpt)],
                        out_hbm.at[core, pl.ds(r0, rpt)])

    return k(hs, src_t, dst_t, zeros_f)


def _tc_matmul(xp, W1):
    rpad, fin = xp.shape
    hid = W1.shape[1]
    nblk = rpad // _BR

    def body(x_ref, w_ref, o_ref):
        o_ref[...] = jnp.dot(x_ref[...], w_ref[...],
                             preferred_element_type=jnp.float32)

    return pl.pallas_call(
        body,
        grid=(nblk,),
        in_specs=[pl.BlockSpec((_BR, fin), lambda i: (i, 0)),
                  pl.BlockSpec((fin, hid), lambda i: (0, 0))],
        out_specs=pl.BlockSpec((_BR, hid), lambda i: (i, 0)),
        out_shape=jax.ShapeDtypeStruct((rpad, hid), jnp.float32),
    )(xp, W1)


def _tc_scale(h, degp):
    rpad, hid = h.shape
    nblk = rpad // _BR

    def body(h_ref, d_ref, o_ref):
        deg = 1.0 + d_ref[0, :, 0:1] + d_ref[1, :, 0:1]
        o_ref[...] = h_ref[...] * lax.rsqrt(deg)

    return pl.pallas_call(
        body,
        grid=(nblk,),
        in_specs=[pl.BlockSpec((_BR, hid), lambda i: (i, 0)),
                  pl.BlockSpec((2, _BR, hid), lambda i: (0, i, 0))],
        out_specs=pl.BlockSpec((_BR, hid), lambda i: (i, 0)),
        out_shape=jax.ShapeDtypeStruct((rpad, hid), jnp.float32),
    )(h, degp)


def _tc_finale(accp, hs, degp, b1r, batchv, batchs, W2p, b2r, nout):
    rpad, hid = hs.shape
    nblk = rpad // _BR

    def body(a_ref, hs_ref, d_ref, b1_ref, bv_ref, bs_ref, w2_ref, b2_ref,
             o_ref, pooled):
        i = pl.program_id(0)

        @pl.when(i == 0)
        def _():
            pooled[...] = jnp.full((_NG, hid), -jnp.inf, jnp.float32)

        deg = 1.0 + d_ref[0, :, 0:1] + d_ref[1, :, 0:1]
        h2 = a_ref[0] + a_ref[1] + hs_ref[...]
        h2 = jnp.maximum(h2 * lax.rsqrt(deg) + b1_ref[0:1, :], 0.0)
        bv = bv_ref[...]
        lo = bs_ref[0, 0, 0]
        hi = jnp.minimum(bs_ref[0, 0, _BR - 1], _NG - 1)

        def seg(g, carry):
            vals = jnp.where(bv == g, h2, -jnp.inf)
            m = jnp.max(vals, axis=0, keepdims=True)
            cur = pooled[pl.ds(g, 1), :]
            pooled[pl.ds(g, 1), :] = jnp.maximum(cur, m)
            return carry

        lax.fori_loop(lo, hi + 1, seg, 0)

        @pl.when(i == nblk - 1)
        def _():
            p = pooled[...]
            p = jnp.where(jnp.isfinite(p), p, 0.0)
            logits = jnp.dot(p, w2_ref[...],
                             preferred_element_type=jnp.float32) + b2_ref[0:1, :]
            lane = lax.broadcasted_iota(jnp.int32, (_NG, hid), 1)
            ok = lane < nout
            neg = jnp.where(ok, logits, -jnp.inf)
            mx = jnp.max(neg, axis=1, keepdims=True)
            ex = jnp.where(ok, jnp.exp(logits - mx), 0.0)
            lse = jnp.log(jnp.sum(ex, axis=1, keepdims=True)) + mx
            o_ref[...] = logits - lse

    return pl.pallas_call(
        body,
        grid=(nblk,),
        in_specs=[
            pl.BlockSpec((2, _BR, hid), lambda i: (0, i, 0)),
            pl.BlockSpec((_BR, hid), lambda i: (i, 0)),
            pl.BlockSpec((2, _BR, hid), lambda i: (0, i, 0)),
            pl.BlockSpec((1, hid), lambda i: (0, 0)),
            pl.BlockSpec((_BR, 1), lambda i: (i, 0)),
            pl.BlockSpec((1, 1, _BR), lambda i: (i, 0, 0),
                         memory_space=pltpu.SMEM),
            pl.BlockSpec((hid, hid), lambda i: (0, 0)),
            pl.BlockSpec((1, hid), lambda i: (0, 0)),
        ],
        out_specs=pl.BlockSpec((_NG, hid), lambda i: (0, 0)),
        out_shape=jax.ShapeDtypeStruct((_NG, hid), jnp.float32),
        scratch_shapes=[pltpu.VMEM((_NG, hid), jnp.float32)],
    )(accp, hs, degp, b1r, batchv, batchs, W2p, b2r)


def kernel(x, edge_index, batch, W1, b1, W2, b2):
    n, fin = x.shape
    hid = W1.shape[1]
    nout = W2.shape[1]
    e = edge_index.shape[1]

    rpad = -(-(n + 1) // _BR) * _BR
    epg = _NTILES * _CHUNK
    ep = -(-e // epg) * epg
    nch = ep // epg

    pad = jnp.full((ep - e,), n, dtype=jnp.int32)
    src_t = jnp.concatenate([edge_index[0], pad]).reshape(
        _NTILES, nch, 1, _CHUNK)
    dst_t = jnp.concatenate([edge_index[1], pad]).reshape(
        _NTILES, nch, 1, _CHUNK)

    ones_w = jnp.ones((_CHUNK, hid), jnp.float32)
    zeros_f = jnp.zeros((rpad, hid), jnp.float32)
    xp = jnp.zeros((rpad, fin), x.dtype).at[:n].set(x)

    degp = _sc_degree(dst_t, ones_w, zeros_f, rpad, hid)
    h = _tc_matmul(xp, W1)
    hs = _tc_scale(h, degp)
    accp = _sc_edge_pass(hs, src_t, dst_t, zeros_f, rpad)

    batchp = jnp.concatenate(
        [batch.astype(jnp.int32), jnp.full((rpad - n,), _NG, jnp.int32)])
    batchv = batchp.reshape(rpad, 1)
    batchs = batchp.reshape(rpad // _BR, 1, _BR)
    W2p = jnp.pad(W2, ((0, 0), (0, hid - nout)))
    b2r = jnp.pad(b2, (0, hid - nout)).reshape(1, hid)
    b1r = b1.reshape(1, hid)

    out = _tc_finale(accp, hs, degp, b1r, batchv, batchs, W2p, b2r, nout)
    return out[:, :nout]
